# baseline (speedup 1.0000x reference)
import dataclasses
import functools

import jax
import jax.numpy as jnp
import numpy as np
from jax import lax
from jax.experimental import pallas as pl
from jax.experimental.pallas import tpu as pltpu
from jax.experimental.pallas import tpu_sc as plsc

N = 10000
E = 320000
NP = 10240
NC, NS = 2, 16
NW = NC * NS
CHUNK = 128
ECH = 2560
CPW = ECH // NW
EPAD = ECH * CHUNK
RPS = NP // NS
NBUF = 2
NGRP = CPW // NBUF

_mesh = plsc.VectorSubcoreMesh(
    core_axis_name="c", subcore_axis_name="s", num_cores=NC, num_subcores=NS
)

_sc_params = pltpu.CompilerParams()
if "needs_layout_passes" in pltpu.CompilerParams.__dataclass_fields__:
    _sc_params = dataclasses.replace(_sc_params, needs_layout_passes=False)


@functools.partial(
    pl.kernel,
    out_type=jax.ShapeDtypeStruct((NW, NP), jnp.float32),
    mesh=_mesh,
    scratch_types=[
        pltpu.VMEM((CPW, CHUNK), jnp.int32),
        pltpu.VMEM((NP,), jnp.float32),
    ],
    compiler_params=_sc_params,
)
def _deg_kernel(dst2_hbm, out_hbm, didx, hist):
    c = lax.axis_index("c")
    s = lax.axis_index("s")
    wid = s * NC + c
    pltpu.sync_copy(dst2_hbm.at[pl.ds(wid * CPW, CPW)], didx)

    @pl.loop(0, NP // 16)
    def _(i):
        hist[pl.ds(i * 16, 16)] = jnp.zeros((16,), jnp.float32)

    ones16 = jnp.ones((16,), jnp.float32)

    @pl.loop(0, CPW)
    def _(i):
        for j in range(CHUNK // 16):
            idxv = didx[i, pl.ds(j * 16, 16)]
            plsc.addupdate_scatter(hist, [idxv], ones16)

    pltpu.sync_copy(hist, out_hbm.at[wid])


def _make_agg(C):
    @functools.partial(
        pl.kernel,
        out_type=jax.ShapeDtypeStruct((NC, NP, C), jnp.float32),
        mesh=_mesh,
        scratch_types=[
            pltpu.VMEM((1, CHUNK), jnp.int32),
            pltpu.VMEM((1, CHUNK), jnp.int32),
            pltpu.VMEM((CHUNK, C), jnp.float32),
            pltpu.VMEM((8, C), jnp.float32),
            pltpu.VMEM_SHARED((NP, C), jnp.float32),
            pltpu.SemaphoreType.DMA,
        ],
    )
    def agg(mp_hbm, src2_hbm, dst2_hbm, out_hbm, sidx, didx, rows, zv, accum,
            sem):
        c = lax.axis_index("c")
        s = lax.axis_index("s")
        wid = s * NC + c

        for i in range(8):
            for j in range(C // 16):
                zv[i, pl.ds(j * 16, 16)] = jnp.zeros((16,), jnp.float32)

        @pl.loop(0, RPS // 8)
        def _(i):
            pltpu.sync_copy(zv, accum.at[pl.ds(s * RPS + i * 8, 8)])

        plsc.subcore_barrier()

        @pl.loop(wid * CPW, (wid + 1) * CPW)
        def _(i):
            pltpu.sync_copy(src2_hbm.at[pl.ds(i, 1)], sidx)
            pltpu.sync_copy(dst2_hbm.at[pl.ds(i, 1)], didx)
            pltpu.async_copy(mp_hbm.at[sidx.at[0]], rows, sem).wait()
            pltpu.sync_copy(rows, accum.at[didx.at[0]], add=True)

        plsc.subcore_barrier()
        pltpu.sync_copy(
            accum.at[pl.ds(s * RPS, RPS)], out_hbm.at[c].at[pl.ds(s * RPS, RPS)]
        )

    return agg


_agg128 = _make_agg(128)


_BLK = 1280


def _mm_body(x_ref, w_ref, o_ref):
    o_ref[...] = jnp.dot(x_ref[...], w_ref[...], preferred_element_type=jnp.float32)


def _tc_mm(x, W):
    cin, cout = W.shape
    return pl.pallas_call(
        _mm_body,
        grid=(NP // _BLK,),
        in_specs=[
            pl.BlockSpec((_BLK, cin), lambda i: (i, 0)),
            pl.BlockSpec((cin, cout), lambda i: (0, 0)),
        ],
        out_specs=pl.BlockSpec((_BLK, cout), lambda i: (i, 0)),
        out_shape=jax.ShapeDtypeStruct((NP, cout), jnp.float32),
    )(x, W)


def _prep_body(degp_ref, m1_ref, dinv_ref, m1p_ref):
    deg = degp_ref[0]
    for k in range(1, NW):
        deg = deg + degp_ref[k]
    dinv = lax.rsqrt(deg + 1.0)
    dinv_ref[...] = dinv
    m1p_ref[...] = m1_ref[...] * dinv


def _tc_prep(degp, m1_raw):
    return pl.pallas_call(
        _prep_body,
        grid=(NP // _BLK,),
        in_specs=[
            pl.BlockSpec((NW, _BLK, 1), lambda i: (0, i, 0)),
            pl.BlockSpec((_BLK, 128), lambda i: (i, 0)),
        ],
        out_specs=[
            pl.BlockSpec((_BLK, 1), lambda i: (i, 0)),
            pl.BlockSpec((_BLK, 128), lambda i: (i, 0)),
        ],
        out_shape=[
            jax.ShapeDtypeStruct((NP, 1), jnp.float32),
            jax.ShapeDtypeStruct((NP, 128), jnp.float32),
        ],
    )(degp, m1_raw)


def _mid_body(p_ref, mp_ref, dinv_ref, b_ref, w_ref, o_ref):
    h = dinv_ref[...] * (p_ref[0] + p_ref[1] + mp_ref[...]) + b_ref[...]
    h = jnp.maximum(h, 0.0)
    o_ref[...] = dinv_ref[...] * jnp.dot(
        h, w_ref[...], preferred_element_type=jnp.float32
    )


def _tc_mid(P, Mp, dinv, b, W):
    cin, cout = W.shape
    return pl.pallas_call(
        _mid_body,
        grid=(NP // _BLK,),
        in_specs=[
            pl.BlockSpec((NC, _BLK, cin), lambda i: (0, i, 0)),
            pl.BlockSpec((_BLK, cin), lambda i: (i, 0)),
            pl.BlockSpec((_BLK, 1), lambda i: (i, 0)),
            pl.BlockSpec((1, cin), lambda i: (0, 0)),
            pl.BlockSpec((cin, cout), lambda i: (0, 0)),
        ],
        out_specs=pl.BlockSpec((_BLK, cout), lambda i: (i, 0)),
        out_shape=jax.ShapeDtypeStruct((NP, cout), jnp.float32),
    )(P, Mp, dinv, b, W)


def _final_body(p_ref, mp_ref, dinv_ref, b_ref, o_ref):
    o_ref[...] = (
        dinv_ref[...] * (p_ref[0] + p_ref[1] + mp_ref[...]) + b_ref[...]
    )


def _tc_final(P, Mp, dinv, b):
    c = Mp.shape[1]
    return pl.pallas_call(
        _final_body,
        grid=(NP // _BLK,),
        in_specs=[
            pl.BlockSpec((NC, _BLK, c), lambda i: (0, i, 0)),
            pl.BlockSpec((_BLK, c), lambda i: (i, 0)),
            pl.BlockSpec((_BLK, 1), lambda i: (i, 0)),
            pl.BlockSpec((1, c), lambda i: (0, 0)),
        ],
        out_specs=pl.BlockSpec((_BLK, c), lambda i: (i, 0)),
        out_shape=jax.ShapeDtypeStruct((NP, c), jnp.float32),
    )(P, Mp, dinv, b)


_PAD_DST = (N + (np.arange(EPAD - E) % (NP - N))).astype(np.int32)


def kernel(x, edge_index, W1, b1, W2, b2, W3, b3):
    ei = edge_index.astype(jnp.int32)
    src2 = jnp.concatenate(
        [ei[0], jnp.zeros((EPAD - E,), jnp.int32)]
    ).reshape(ECH, CHUNK)
    dst2 = jnp.concatenate([ei[1], _PAD_DST]).reshape(ECH, CHUNK)
    x_p = jnp.pad(x, ((0, NP - N), (0, 0)))

    W2p = jnp.pad(W2, ((0, 0), (0, 128 - W2.shape[1])))
    W3p = jnp.pad(W3, ((0, 128 - W3.shape[0]), (0, 128 - W3.shape[1])))
    b2p = jnp.pad(b2, (0, 128 - b2.shape[0])).reshape(1, -1)
    b3p = jnp.pad(b3, (0, 128 - b3.shape[0])).reshape(1, -1)

    degp = _deg_kernel(dst2)
    m1_raw = _tc_mm(x_p, W1)
    dinv, m1p = _tc_prep(degp.reshape(NW, NP, 1), m1_raw)
    p1 = _agg128(m1p, src2, dst2)
    m2p = _tc_mid(p1, m1p, dinv, b1.reshape(1, -1), W2p)
    p2 = _agg128(m2p, src2, dst2)
    m3p = _tc_mid(p2, m2p, dinv, b2p, W3p)
    p3 = _agg128(m3p, src2, dst2)
    out = _tc_final(p3, m3p, dinv, b3p)
    return out[:N, : W3.shape[1]]


# speedup vs baseline: 1.9830x; 1.9830x over previous
import dataclasses
import functools

import jax
import jax.numpy as jnp
import numpy as np
from jax import lax
from jax.experimental import pallas as pl
from jax.experimental.pallas import tpu as pltpu
from jax.experimental.pallas import tpu_sc as plsc

N = 10000
E = 320000
NP = 10240
NC, NS = 2, 16
NW = NC * NS
CHUNK = 128
ECH = 2560
CPW = ECH // NW
EPAD = ECH * CHUNK
RPS = NP // NS
NBUF = 2
NGRP = CPW // NBUF

_mesh = plsc.VectorSubcoreMesh(
    core_axis_name="c", subcore_axis_name="s", num_cores=NC, num_subcores=NS
)

_sc_params = pltpu.CompilerParams()
if "needs_layout_passes" in pltpu.CompilerParams.__dataclass_fields__:
    _sc_params = dataclasses.replace(_sc_params, needs_layout_passes=False)


@functools.partial(
    pl.kernel,
    out_type=jax.ShapeDtypeStruct((NW, NP), jnp.float32),
    mesh=_mesh,
    scratch_types=[
        pltpu.VMEM((CPW, CHUNK), jnp.int32),
        pltpu.VMEM((NP,), jnp.float32),
    ],
    compiler_params=_sc_params,
)
def _deg_kernel(dst2_hbm, out_hbm, didx, hist):
    c = lax.axis_index("c")
    s = lax.axis_index("s")
    wid = s * NC + c
    pltpu.sync_copy(dst2_hbm.at[pl.ds(wid * CPW, CPW)], didx)

    @pl.loop(0, NP // 16)
    def _(i):
        hist[pl.ds(i * 16, 16)] = jnp.zeros((16,), jnp.float32)

    ones16 = jnp.ones((16,), jnp.float32)

    @pl.loop(0, CPW)
    def _(i):
        for j in range(CHUNK // 16):
            idxv = didx[i, pl.ds(j * 16, 16)]
            plsc.addupdate_scatter(hist, [idxv], ones16)

    pltpu.sync_copy(hist, out_hbm.at[wid])


def _make_agg(C):
    @functools.partial(
        pl.kernel,
        out_type=jax.ShapeDtypeStruct((NC, NP, C), jnp.float32),
        mesh=_mesh,
        scratch_types=[
            pltpu.VMEM((1, CHUNK), jnp.int32),
            pltpu.VMEM((1, CHUNK), jnp.int32),
            pltpu.VMEM((CHUNK, C), jnp.float32),
            pltpu.VMEM((8, C), jnp.float32),
            pltpu.VMEM_SHARED((NP, C), jnp.float32),
            pltpu.SemaphoreType.DMA,
        ],
    )
    def agg(mp_hbm, src2_hbm, dst2_hbm, out_hbm, sidx, didx, rows, zv, accum,
            sem):
        c = lax.axis_index("c")
        s = lax.axis_index("s")
        wid = s * NC + c

        for i in range(8):
            for j in range(C // 16):
                zv[i, pl.ds(j * 16, 16)] = jnp.zeros((16,), jnp.float32)

        @pl.loop(0, RPS // 8)
        def _(i):
            pltpu.sync_copy(zv, accum.at[pl.ds(s * RPS + i * 8, 8)])

        plsc.subcore_barrier()

        @pl.loop(wid * CPW, (wid + 1) * CPW)
        def _(i):
            pltpu.sync_copy(src2_hbm.at[pl.ds(i, 1)], sidx)
            pltpu.sync_copy(dst2_hbm.at[pl.ds(i, 1)], didx)
            pltpu.async_copy(mp_hbm.at[sidx.at[0]], rows, sem).wait()
            pltpu.sync_copy(rows, accum.at[didx.at[0]], add=True)

        plsc.subcore_barrier()
        pltpu.sync_copy(
            accum.at[pl.ds(s * RPS, RPS)], out_hbm.at[c].at[pl.ds(s * RPS, RPS)]
        )

    return agg


_agg128 = _make_agg(128)


_BLK = 1280


def _mm_body(x_ref, w_ref, o_ref):
    o_ref[...] = jnp.dot(x_ref[...], w_ref[...], preferred_element_type=jnp.float32)


def _tc_mm(x, W):
    cin, cout = W.shape
    return pl.pallas_call(
        _mm_body,
        grid=(NP // _BLK,),
        in_specs=[
            pl.BlockSpec((_BLK, cin), lambda i: (i, 0)),
            pl.BlockSpec((cin, cout), lambda i: (0, 0)),
        ],
        out_specs=pl.BlockSpec((_BLK, cout), lambda i: (i, 0)),
        out_shape=jax.ShapeDtypeStruct((NP, cout), jnp.float32),
    )(x, W)


def _prep_body(degp_ref, m1_ref, dinv_ref, m1p_ref):
    deg = degp_ref[0]
    for k in range(1, NW):
        deg = deg + degp_ref[k]
    dinv = lax.rsqrt(deg + 1.0)
    dinv_ref[...] = dinv
    m1p_ref[...] = m1_ref[...] * dinv


def _tc_prep(degp, m1_raw):
    return pl.pallas_call(
        _prep_body,
        grid=(NP // _BLK,),
        in_specs=[
            pl.BlockSpec((NW, _BLK, 1), lambda i: (0, i, 0)),
            pl.BlockSpec((_BLK, 128), lambda i: (i, 0)),
        ],
        out_specs=[
            pl.BlockSpec((_BLK, 1), lambda i: (i, 0)),
            pl.BlockSpec((_BLK, 128), lambda i: (i, 0)),
        ],
        out_shape=[
            jax.ShapeDtypeStruct((NP, 1), jnp.float32),
            jax.ShapeDtypeStruct((NP, 128), jnp.float32),
        ],
    )(degp, m1_raw)


def _mid_body(p_ref, mp_ref, dinv_ref, b_ref, w_ref, o_ref):
    h = dinv_ref[...] * (p_ref[0] + p_ref[1] + mp_ref[...]) + b_ref[...]
    h = jnp.maximum(h, 0.0)
    o_ref[...] = dinv_ref[...] * jnp.dot(
        h, w_ref[...], preferred_element_type=jnp.float32
    )


def _tc_mid(P, Mp, dinv, b, W):
    cin, cout = W.shape
    return pl.pallas_call(
        _mid_body,
        grid=(NP // _BLK,),
        in_specs=[
            pl.BlockSpec((NC, _BLK, cin), lambda i: (0, i, 0)),
            pl.BlockSpec((_BLK, cin), lambda i: (i, 0)),
            pl.BlockSpec((_BLK, 1), lambda i: (i, 0)),
            pl.BlockSpec((1, cin), lambda i: (0, 0)),
            pl.BlockSpec((cin, cout), lambda i: (0, 0)),
        ],
        out_specs=pl.BlockSpec((_BLK, cout), lambda i: (i, 0)),
        out_shape=jax.ShapeDtypeStruct((NP, cout), jnp.float32),
    )(P, Mp, dinv, b, W)


def _final_body(p_ref, mp_ref, dinv_ref, b_ref, o_ref):
    o_ref[...] = (
        dinv_ref[...] * (p_ref[0] + p_ref[1] + mp_ref[...]) + b_ref[...]
    )


def _tc_final(P, Mp, dinv, b):
    c = Mp.shape[1]
    return pl.pallas_call(
        _final_body,
        grid=(NP // _BLK,),
        in_specs=[
            pl.BlockSpec((NC, _BLK, c), lambda i: (0, i, 0)),
            pl.BlockSpec((_BLK, c), lambda i: (i, 0)),
            pl.BlockSpec((_BLK, 1), lambda i: (i, 0)),
            pl.BlockSpec((1, c), lambda i: (0, 0)),
        ],
        out_specs=pl.BlockSpec((_BLK, c), lambda i: (i, 0)),
        out_shape=jax.ShapeDtypeStruct((NP, c), jnp.float32),
    )(P, Mp, dinv, b)


PADW = (EPAD - E) // NW
_PAD_SRC = ((np.arange(NW * PADW).reshape(NW, PADW) * 131) % N).astype(np.int32)
_PAD_DST = (N + (np.arange(NW * PADW).reshape(NW, PADW) % (NP - N))).astype(
    np.int32
)


def kernel(x, edge_index, W1, b1, W2, b2, W3, b3):
    ei = edge_index.astype(jnp.int32)
    src2 = jnp.concatenate(
        [ei[0].reshape(NW, E // NW), jnp.asarray(_PAD_SRC)], axis=1
    ).reshape(ECH, CHUNK)
    dst2 = jnp.concatenate(
        [ei[1].reshape(NW, E // NW), jnp.asarray(_PAD_DST)], axis=1
    ).reshape(ECH, CHUNK)
    x_p = jnp.pad(x, ((0, NP - N), (0, 0)))

    W2p = jnp.pad(W2, ((0, 0), (0, 128 - W2.shape[1])))
    W3p = jnp.pad(W3, ((0, 128 - W3.shape[0]), (0, 128 - W3.shape[1])))
    b2p = jnp.pad(b2, (0, 128 - b2.shape[0])).reshape(1, -1)
    b3p = jnp.pad(b3, (0, 128 - b3.shape[0])).reshape(1, -1)

    degp = _deg_kernel(dst2)
    m1_raw = _tc_mm(x_p, W1)
    dinv, m1p = _tc_prep(degp.reshape(NW, NP, 1), m1_raw)
    p1 = _agg128(m1p, src2, dst2)
    m2p = _tc_mid(p1, m1p, dinv, b1.reshape(1, -1), W2p)
    p2 = _agg128(m2p, src2, dst2)
    m3p = _tc_mid(p2, m2p, dinv, b2p, W3p)
    p3 = _agg128(m3p, src2, dst2)
    out = _tc_final(p3, m3p, dinv, b3p)
    return out[:N, : W3.shape[1]]


# speedup vs baseline: 3.1293x; 1.5781x over previous
import dataclasses
import functools

import jax
import jax.numpy as jnp
import numpy as np
from jax import lax
from jax.experimental import pallas as pl
from jax.experimental.pallas import tpu as pltpu
from jax.experimental.pallas import tpu_sc as plsc

N = 10000
E = 320000
NP = 10240
NC, NS = 2, 16
NW = NC * NS
CHUNK = 128
ECH = 2560
CPW = ECH // NW
EPAD = ECH * CHUNK
RPS = NP // NS
NBUF = 2
NGRP = CPW // NBUF

_mesh = plsc.VectorSubcoreMesh(
    core_axis_name="c", subcore_axis_name="s", num_cores=NC, num_subcores=NS
)

_sc_params = pltpu.CompilerParams()
if "needs_layout_passes" in pltpu.CompilerParams.__dataclass_fields__:
    _sc_params = dataclasses.replace(_sc_params, needs_layout_passes=False)


@functools.partial(
    pl.kernel,
    out_type=jax.ShapeDtypeStruct((NW, NP), jnp.float32),
    mesh=_mesh,
    scratch_types=[
        pltpu.VMEM((CPW, CHUNK), jnp.int32),
        pltpu.VMEM((NP,), jnp.float32),
    ],
    compiler_params=_sc_params,
)
def _deg_kernel(dst2_hbm, out_hbm, didx, hist):
    c = lax.axis_index("c")
    s = lax.axis_index("s")
    wid = s * NC + c
    pltpu.sync_copy(dst2_hbm.at[pl.ds(wid * CPW, CPW)], didx)

    @pl.loop(0, NP // 16)
    def _(i):
        hist[pl.ds(i * 16, 16)] = jnp.zeros((16,), jnp.float32)

    ones16 = jnp.ones((16,), jnp.float32)

    @pl.loop(0, CPW)
    def _(i):
        for j in range(CHUNK // 16):
            idxv = didx[i, pl.ds(j * 16, 16)]
            plsc.addupdate_scatter(hist, [idxv], ones16)

    pltpu.sync_copy(hist, out_hbm.at[wid])


def _make_agg(C, tc_tiling=True):
    params = _sc_params
    if not tc_tiling:
        params = dataclasses.replace(params, use_tc_tiling_on_sc=False)

    @functools.partial(
        pl.kernel,
        out_type=jax.ShapeDtypeStruct((NC, NP, C), jnp.float32),
        mesh=_mesh,
        scratch_types=[
            [pltpu.VMEM((1, CHUNK), jnp.int32) for _ in range(4)],
            [pltpu.VMEM((1, CHUNK), jnp.int32) for _ in range(4)],
            [pltpu.VMEM((CHUNK, C), jnp.float32) for _ in range(2)],
            pltpu.VMEM((8, C), jnp.float32),
            pltpu.VMEM_SHARED((NP, C), jnp.float32),
            [pltpu.SemaphoreType.DMA for _ in range(2)],
            [pltpu.SemaphoreType.DMA for _ in range(2)],
            [pltpu.SemaphoreType.DMA for _ in range(4)],
        ],
        compiler_params=params,
    )
    def agg(mp_hbm, src2_hbm, dst2_hbm, out_hbm, sidxb, didxb, rows, zv,
            accum, gsem, ssem, stsem):
        c = lax.axis_index("c")
        s = lax.axis_index("s")
        wid = s * NC + c
        base = wid * CPW

        def stage(j, q):
            pltpu.async_copy(src2_hbm.at[pl.ds(base + j, 1)], sidxb[q],
                             stsem[q])
            pltpu.async_copy(dst2_hbm.at[pl.ds(base + j, 1)], didxb[q],
                             stsem[q])

        def wait_stage(q):
            for _ in range(2):
                pltpu.make_async_copy(
                    src2_hbm.at[pl.ds(base, 1)], sidxb[q], stsem[q]).wait()

        def gather(q, b):
            pltpu.async_copy(mp_hbm.at[sidxb[q].at[0]], rows[b], gsem[b])

        def wait_gather(b):
            pltpu.make_async_copy(
                mp_hbm.at[sidxb[0].at[0]], rows[b], gsem[b]).wait()

        def scatter(q, b):
            pltpu.async_copy(rows[b], accum.at[didxb[q].at[0]], ssem[b],
                             add=True)

        def wait_scatter(b):
            pltpu.make_async_copy(
                rows[b], accum.at[didxb[0].at[0]], ssem[b]).wait()

        for i in range(8):
            for j in range(C // 16):
                zv[i, pl.ds(j * 16, 16)] = jnp.zeros((16,), jnp.float32)

        @pl.loop(0, RPS // 8)
        def _(i):
            pltpu.sync_copy(zv, accum.at[pl.ds(s * RPS + i * 8, 8)])

        for q in range(4):
            stage(q, q)
        wait_stage(0)
        gather(0, 0)
        wait_stage(1)
        gather(1, 1)

        plsc.subcore_barrier()

        wait_gather(0)
        scatter(0, 0)
        wait_stage(2)
        wait_gather(1)
        scatter(1, 1)
        wait_scatter(0)
        stage(4, 0)
        gather(2, 0)
        wait_stage(3)
        wait_gather(0)
        scatter(2, 0)
        wait_scatter(1)
        stage(5, 1)
        gather(3, 1)

        NT = (CPW - 4) // 4

        @pl.loop(0, NT)
        def _(t):
            j0 = t * 4 + 4
            for k in range(4):
                b = k % 2
                wait_stage(k)
                wait_gather(1 - b)
                scatter((k - 1) % 4, 1 - b)
                wait_scatter(b)
                if k < 2:
                    stage(j0 + k + 2, (k + 2) % 4)
                else:
                    @pl.when(t < NT - 1)
                    def _(sr=j0 + k + 2, qq=(k + 2) % 4):
                        stage(sr, qq)
                gather(k, b)

        wait_gather(1)
        scatter(3, 1)
        wait_scatter(0)
        wait_scatter(1)

        plsc.subcore_barrier()
        pltpu.sync_copy(
            accum.at[pl.ds(s * RPS, RPS)], out_hbm.at[c].at[pl.ds(s * RPS, RPS)]
        )

    return agg


_agg128 = _make_agg(128)


_BLK = 1280


def _mm_body(x_ref, w_ref, o_ref):
    o_ref[...] = jnp.dot(x_ref[...], w_ref[...], preferred_element_type=jnp.float32)


def _tc_mm(x, W):
    cin, cout = W.shape
    return pl.pallas_call(
        _mm_body,
        grid=(NP // _BLK,),
        in_specs=[
            pl.BlockSpec((_BLK, cin), lambda i: (i, 0)),
            pl.BlockSpec((cin, cout), lambda i: (0, 0)),
        ],
        out_specs=pl.BlockSpec((_BLK, cout), lambda i: (i, 0)),
        out_shape=jax.ShapeDtypeStruct((NP, cout), jnp.float32),
    )(x, W)


def _prep_body(degp_ref, m1_ref, dinv_ref, m1p_ref):
    deg = degp_ref[0]
    for k in range(1, NW):
        deg = deg + degp_ref[k]
    dinv = lax.rsqrt(deg + 1.0)
    dinv_ref[...] = dinv
    m1p_ref[...] = m1_ref[...] * dinv


def _tc_prep(degp, m1_raw):
    return pl.pallas_call(
        _prep_body,
        grid=(NP // _BLK,),
        in_specs=[
            pl.BlockSpec((NW, _BLK, 1), lambda i: (0, i, 0)),
            pl.BlockSpec((_BLK, 128), lambda i: (i, 0)),
        ],
        out_specs=[
            pl.BlockSpec((_BLK, 1), lambda i: (i, 0)),
            pl.BlockSpec((_BLK, 128), lambda i: (i, 0)),
        ],
        out_shape=[
            jax.ShapeDtypeStruct((NP, 1), jnp.float32),
            jax.ShapeDtypeStruct((NP, 128), jnp.float32),
        ],
    )(degp, m1_raw)


def _mid_body(p_ref, mp_ref, dinv_ref, b_ref, w_ref, o_ref):
    h = dinv_ref[...] * (p_ref[0] + p_ref[1] + mp_ref[...]) + b_ref[...]
    h = jnp.maximum(h, 0.0)
    o_ref[...] = dinv_ref[...] * jnp.dot(
        h, w_ref[...], preferred_element_type=jnp.float32
    )


def _tc_mid(P, Mp, dinv, b, W):
    cin, cout = W.shape
    return pl.pallas_call(
        _mid_body,
        grid=(NP // _BLK,),
        in_specs=[
            pl.BlockSpec((NC, _BLK, cin), lambda i: (0, i, 0)),
            pl.BlockSpec((_BLK, cin), lambda i: (i, 0)),
            pl.BlockSpec((_BLK, 1), lambda i: (i, 0)),
            pl.BlockSpec((1, cin), lambda i: (0, 0)),
            pl.BlockSpec((cin, cout), lambda i: (0, 0)),
        ],
        out_specs=pl.BlockSpec((_BLK, cout), lambda i: (i, 0)),
        out_shape=jax.ShapeDtypeStruct((NP, cout), jnp.float32),
    )(P, Mp, dinv, b, W)


def _final_body(p_ref, mp_ref, dinv_ref, b_ref, o_ref):
    o_ref[...] = (
        dinv_ref[...] * (p_ref[0] + p_ref[1] + mp_ref[...]) + b_ref[...]
    )


def _tc_final(P, Mp, dinv, b):
    c = Mp.shape[1]
    return pl.pallas_call(
        _final_body,
        grid=(NP // _BLK,),
        in_specs=[
            pl.BlockSpec((NC, _BLK, c), lambda i: (0, i, 0)),
            pl.BlockSpec((_BLK, c), lambda i: (i, 0)),
            pl.BlockSpec((_BLK, 1), lambda i: (i, 0)),
            pl.BlockSpec((1, c), lambda i: (0, 0)),
        ],
        out_specs=pl.BlockSpec((_BLK, c), lambda i: (i, 0)),
        out_shape=jax.ShapeDtypeStruct((NP, c), jnp.float32),
    )(P, Mp, dinv, b)


PADW = (EPAD - E) // NW
_PAD_SRC = ((np.arange(NW * PADW).reshape(NW, PADW) * 131) % N).astype(np.int32)
_PAD_DST = (N + (np.arange(NW * PADW).reshape(NW, PADW) % (NP - N))).astype(
    np.int32
)


def kernel(x, edge_index, W1, b1, W2, b2, W3, b3):
    ei = edge_index.astype(jnp.int32)
    src2 = jnp.concatenate(
        [ei[0].reshape(NW, E // NW), jnp.asarray(_PAD_SRC)], axis=1
    ).reshape(ECH, CHUNK)
    dst2 = jnp.concatenate(
        [ei[1].reshape(NW, E // NW), jnp.asarray(_PAD_DST)], axis=1
    ).reshape(ECH, CHUNK)
    x_p = jnp.pad(x, ((0, NP - N), (0, 0)))

    W2p = jnp.pad(W2, ((0, 0), (0, 128 - W2.shape[1])))
    W3p = jnp.pad(W3, ((0, 128 - W3.shape[0]), (0, 128 - W3.shape[1])))
    b2p = jnp.pad(b2, (0, 128 - b2.shape[0])).reshape(1, -1)
    b3p = jnp.pad(b3, (0, 128 - b3.shape[0])).reshape(1, -1)

    degp = _deg_kernel(dst2)
    m1_raw = _tc_mm(x_p, W1)
    dinv, m1p = _tc_prep(degp.reshape(NW, NP, 1), m1_raw)
    p1 = _agg128(m1p, src2, dst2)
    m2p = _tc_mid(p1, m1p, dinv, b1.reshape(1, -1), W2p)
    p2 = _agg128(m2p, src2, dst2)
    m3p = _tc_mid(p2, m2p, dinv, b2p, W3p)
    p3 = _agg128(m3p, src2, dst2)
    out = _tc_final(p3, m3p, dinv, b3p)
    return out[:N, : W3.shape[1]]


# speedup vs baseline: 3.1578x; 1.0091x over previous
import dataclasses
import functools

import jax
import jax.numpy as jnp
import numpy as np
from jax import lax
from jax.experimental import pallas as pl
from jax.experimental.pallas import tpu as pltpu
from jax.experimental.pallas import tpu_sc as plsc

N = 10000
E = 320000
NP = 10240
NC, NS = 2, 16
NW = NC * NS
CHUNK = 128
ECH = 2560
CPW = ECH // NW
EPAD = ECH * CHUNK
RPS = NP // NS
NBUF = 2
NGRP = CPW // NBUF

_mesh = plsc.VectorSubcoreMesh(
    core_axis_name="c", subcore_axis_name="s", num_cores=NC, num_subcores=NS
)

_sc_params = pltpu.CompilerParams()
if "needs_layout_passes" in pltpu.CompilerParams.__dataclass_fields__:
    _sc_params = dataclasses.replace(_sc_params, needs_layout_passes=False)


@functools.partial(
    pl.kernel,
    out_type=jax.ShapeDtypeStruct((NW, NP), jnp.float32),
    mesh=_mesh,
    scratch_types=[
        pltpu.VMEM((CPW, CHUNK), jnp.int32),
        pltpu.VMEM((NP,), jnp.float32),
    ],
    compiler_params=_sc_params,
)
def _deg_kernel(dst2_hbm, out_hbm, didx, hist):
    c = lax.axis_index("c")
    s = lax.axis_index("s")
    wid = s * NC + c
    pltpu.sync_copy(dst2_hbm.at[pl.ds(wid * CPW, CPW)], didx)

    @pl.loop(0, NP // 16)
    def _(i):
        hist[pl.ds(i * 16, 16)] = jnp.zeros((16,), jnp.float32)

    ones16 = jnp.ones((16,), jnp.float32)

    @pl.loop(0, CPW)
    def _(i):
        for j in range(CHUNK // 16):
            idxv = didx[i, pl.ds(j * 16, 16)]
            plsc.addupdate_scatter(hist, [idxv], ones16)

    pltpu.sync_copy(hist, out_hbm.at[wid])


def _make_agg(C, tc_tiling=True):
    params = _sc_params
    if not tc_tiling:
        params = dataclasses.replace(params, use_tc_tiling_on_sc=False)

    @functools.partial(
        pl.kernel,
        out_type=jax.ShapeDtypeStruct((NC, NP, C), jnp.float32),
        mesh=_mesh,
        scratch_types=[
            [pltpu.VMEM((1, CHUNK), jnp.int32) for _ in range(4)],
            [pltpu.VMEM((1, CHUNK), jnp.int32) for _ in range(4)],
            [pltpu.VMEM((CHUNK, C), jnp.float32) for _ in range(2)],
            pltpu.VMEM((8, C), jnp.float32),
            pltpu.VMEM_SHARED((NP, C), jnp.float32),
            [pltpu.SemaphoreType.DMA for _ in range(2)],
            [pltpu.SemaphoreType.DMA for _ in range(2)],
            [pltpu.SemaphoreType.DMA for _ in range(4)],
        ],
        compiler_params=params,
    )
    def agg(mp_hbm, src2_hbm, dst2_hbm, out_hbm, sidxb, didxb, rows, zv,
            accum, gsem, ssem, stsem):
        c = lax.axis_index("c")
        s = lax.axis_index("s")
        wid = s * NC + c
        base = wid * CPW

        def stage(j, q):
            pltpu.async_copy(src2_hbm.at[pl.ds(base + j, 1)], sidxb[q],
                             stsem[q])
            pltpu.async_copy(dst2_hbm.at[pl.ds(base + j, 1)], didxb[q],
                             stsem[q])

        def wait_stage(q):
            for _ in range(2):
                pltpu.make_async_copy(
                    src2_hbm.at[pl.ds(base, 1)], sidxb[q], stsem[q]).wait()

        def gather(q, b):
            pltpu.async_copy(mp_hbm.at[sidxb[q].at[0]], rows[b], gsem[b])

        def wait_gather(b):
            pltpu.make_async_copy(
                mp_hbm.at[sidxb[0].at[0]], rows[b], gsem[b]).wait()

        def scatter(q, b):
            pltpu.async_copy(rows[b], accum.at[didxb[q].at[0]], ssem[b],
                             add=True)

        def wait_scatter(b):
            pltpu.make_async_copy(
                rows[b], accum.at[didxb[0].at[0]], ssem[b]).wait()

        for i in range(8):
            for j in range(C // 16):
                zv[i, pl.ds(j * 16, 16)] = jnp.zeros((16,), jnp.float32)

        @pl.loop(0, RPS // 8)
        def _(i):
            pltpu.sync_copy(zv, accum.at[pl.ds(s * RPS + i * 8, 8)])

        for q in range(4):
            stage(q, q)
        wait_stage(0)
        gather(0, 0)
        wait_stage(1)
        gather(1, 1)

        plsc.subcore_barrier()

        wait_gather(0)
        scatter(0, 0)
        wait_stage(2)
        wait_gather(1)
        scatter(1, 1)
        wait_scatter(0)
        stage(4, 0)
        gather(2, 0)
        wait_stage(3)
        wait_gather(0)
        scatter(2, 0)
        wait_scatter(1)
        stage(5, 1)
        gather(3, 1)

        NT = (CPW - 4) // 4

        @pl.loop(0, NT)
        def _(t):
            j0 = t * 4 + 4
            for k in range(4):
                b = k % 2
                wait_stage(k)
                wait_gather(1 - b)
                scatter((k - 1) % 4, 1 - b)
                wait_scatter(b)
                if k < 2:
                    stage(j0 + k + 2, (k + 2) % 4)
                else:
                    @pl.when(t < NT - 1)
                    def _(sr=j0 + k + 2, qq=(k + 2) % 4):
                        stage(sr, qq)
                gather(k, b)

        wait_gather(1)
        scatter(3, 1)
        wait_scatter(0)
        wait_scatter(1)

        plsc.subcore_barrier()
        pltpu.sync_copy(
            accum.at[pl.ds(s * RPS, RPS)], out_hbm.at[c].at[pl.ds(s * RPS, RPS)]
        )

    return agg


_agg128 = _make_agg(128)


_BLK = 1280


def _mm_body(x_ref, w_ref, o_ref):
    o_ref[...] = jnp.dot(x_ref[...], w_ref[...], preferred_element_type=jnp.float32)


def _tc_mm(x, W):
    cin, cout = W.shape
    return pl.pallas_call(
        _mm_body,
        grid=(NP // _BLK,),
        in_specs=[
            pl.BlockSpec((_BLK, cin), lambda i: (i, 0)),
            pl.BlockSpec((cin, cout), lambda i: (0, 0)),
        ],
        out_specs=pl.BlockSpec((_BLK, cout), lambda i: (i, 0)),
        out_shape=jax.ShapeDtypeStruct((NP, cout), jnp.float32),
    )(x, W)


def _prep_body(degp_ref, x_ref, w_ref, dinv_ref, m1p_ref):
    deg = degp_ref[0]
    for k in range(1, NW):
        deg = deg + degp_ref[k]
    dinv = lax.rsqrt(deg + 1.0)
    dinv_ref[...] = dinv
    m1p_ref[...] = dinv * jnp.dot(
        x_ref[...], w_ref[...], preferred_element_type=jnp.float32
    )


def _tc_prep(degp, x_p, W1):
    return pl.pallas_call(
        _prep_body,
        grid=(NP // _BLK,),
        in_specs=[
            pl.BlockSpec((NW, _BLK, 1), lambda i: (0, i, 0)),
            pl.BlockSpec((_BLK, 128), lambda i: (i, 0)),
            pl.BlockSpec((128, 128), lambda i: (0, 0)),
        ],
        out_specs=[
            pl.BlockSpec((_BLK, 1), lambda i: (i, 0)),
            pl.BlockSpec((_BLK, 128), lambda i: (i, 0)),
        ],
        out_shape=[
            jax.ShapeDtypeStruct((NP, 1), jnp.float32),
            jax.ShapeDtypeStruct((NP, 128), jnp.float32),
        ],
    )(degp, x_p, W1)


def _mid_body(p_ref, mp_ref, dinv_ref, b_ref, w_ref, o_ref):
    h = dinv_ref[...] * (p_ref[0] + p_ref[1] + mp_ref[...]) + b_ref[...]
    h = jnp.maximum(h, 0.0)
    o_ref[...] = dinv_ref[...] * jnp.dot(
        h, w_ref[...], preferred_element_type=jnp.float32
    )


def _tc_mid(P, Mp, dinv, b, W):
    cin, cout = W.shape
    return pl.pallas_call(
        _mid_body,
        grid=(NP // _BLK,),
        in_specs=[
            pl.BlockSpec((NC, _BLK, cin), lambda i: (0, i, 0)),
            pl.BlockSpec((_BLK, cin), lambda i: (i, 0)),
            pl.BlockSpec((_BLK, 1), lambda i: (i, 0)),
            pl.BlockSpec((1, cin), lambda i: (0, 0)),
            pl.BlockSpec((cin, cout), lambda i: (0, 0)),
        ],
        out_specs=pl.BlockSpec((_BLK, cout), lambda i: (i, 0)),
        out_shape=jax.ShapeDtypeStruct((NP, cout), jnp.float32),
    )(P, Mp, dinv, b, W)


def _final_body(p_ref, mp_ref, dinv_ref, b_ref, o_ref):
    o_ref[...] = (
        dinv_ref[...] * (p_ref[0] + p_ref[1] + mp_ref[...]) + b_ref[...]
    )


def _tc_final(P, Mp, dinv, b):
    c = Mp.shape[1]
    return pl.pallas_call(
        _final_body,
        grid=(NP // _BLK,),
        in_specs=[
            pl.BlockSpec((NC, _BLK, c), lambda i: (0, i, 0)),
            pl.BlockSpec((_BLK, c), lambda i: (i, 0)),
            pl.BlockSpec((_BLK, 1), lambda i: (i, 0)),
            pl.BlockSpec((1, c), lambda i: (0, 0)),
        ],
        out_specs=pl.BlockSpec((_BLK, c), lambda i: (i, 0)),
        out_shape=jax.ShapeDtypeStruct((NP, c), jnp.float32),
    )(P, Mp, dinv, b)


PADW = (EPAD - E) // NW
_PAD_SRC = ((np.arange(NW * PADW).reshape(NW, PADW) * 131) % N).astype(np.int32)
_PAD_DST = (N + (np.arange(NW * PADW).reshape(NW, PADW) % (NP - N))).astype(
    np.int32
)


def kernel(x, edge_index, W1, b1, W2, b2, W3, b3):
    ei = edge_index.astype(jnp.int32)
    src2 = jnp.concatenate(
        [ei[0].reshape(NW, E // NW), jnp.asarray(_PAD_SRC)], axis=1
    ).reshape(ECH, CHUNK)
    dst2 = jnp.concatenate(
        [ei[1].reshape(NW, E // NW), jnp.asarray(_PAD_DST)], axis=1
    ).reshape(ECH, CHUNK)
    x_p = jnp.pad(x, ((0, NP - N), (0, 0)))

    W2p = jnp.pad(W2, ((0, 0), (0, 128 - W2.shape[1])))
    W3p = jnp.pad(W3, ((0, 128 - W3.shape[0]), (0, 128 - W3.shape[1])))
    b2p = jnp.pad(b2, (0, 128 - b2.shape[0])).reshape(1, -1)
    b3p = jnp.pad(b3, (0, 128 - b3.shape[0])).reshape(1, -1)

    degp = _deg_kernel(dst2)
    dinv, m1p = _tc_prep(degp.reshape(NW, NP, 1), x_p, W1)
    p1 = _agg128(m1p, src2, dst2)
    m2p = _tc_mid(p1, m1p, dinv, b1.reshape(1, -1), W2p)
    p2 = _agg128(m2p, src2, dst2)
    m3p = _tc_mid(p2, m2p, dinv, b2p, W3p)
    p3 = _agg128(m3p, src2, dst2)
    out = _tc_final(p3, m3p, dinv, b3p)
    return out[:N, : W3.shape[1]]
